# baseline (speedup 1.0000x reference)
import functools

import jax
import jax.numpy as jnp
from jax import lax
from jax.experimental import pallas as pl
from jax.experimental.pallas import tpu as pltpu
from jax.experimental.pallas import tpu_sc as plsc

_NUM_VOCAB = 1000000
_MAXLEN = 200
_NUM_HID = 64
_BATCH = 4096
_SEQ = 200

_NC = 2
_NS = 16
_NW = _NC * _NS
_TOTAL = _BATCH * _SEQ
_ROWS_PER_W = _TOTAL // _NW
_CHUNK = 2 * _MAXLEN
_NCHUNK = _ROWS_PER_W // _CHUNK
_REPS = _CHUNK // _MAXLEN
_LANES = 16
_SLICES = _NUM_HID // _LANES
_PAIRS = _CHUNK // 2


def _body(idx2_hbm, hoff_hbm, emb2_hbm, pos_hbm, out_hbm,
          pos_v, idx0, idx1, hof0, hof1, raw0, raw1, g0, g1, o0, o1):
    cid = lax.axis_index("c")
    sid = lax.axis_index("s")
    wid = sid * _NC + cid
    base = pl.multiple_of(wid * _ROWS_PER_W, _CHUNK)

    pltpu.sync_copy(pos_hbm, pos_v)

    bufs = ((idx0, hof0, raw0, g0, o0), (idx1, hof1, raw1, g1, o1))

    def gstart(g, idx_v, hof_v, raw_v, gsem):
        off = pl.multiple_of(base + g * _CHUNK, _CHUNK)
        pltpu.sync_copy(idx2_hbm.at[pl.ds(off, _CHUNK)], idx_v)
        pltpu.sync_copy(hoff_hbm.at[pl.ds(off, _CHUNK)],
                        hof_v.at[pl.ds(0, _CHUNK)])
        pltpu.async_copy(emb2_hbm.at[idx_v], raw_v, gsem)

    def gwait(idx_v, raw_v, gsem):
        pltpu.make_async_copy(emb2_hbm.at[idx_v], raw_v, gsem).wait()

    def compact(raw_v, hof_v):
        for rep in range(_REPS):
            def srow(q, carry):
                p = rep * (_MAXLEN // 2) + q
                hv = hof_v[pl.ds(2 * p, _LANES)]
                h0 = pl.multiple_of(hv[0], _NUM_HID)
                h1 = pl.multiple_of(hv[1], _NUM_HID)
                for c in range(_SLICES):
                    pv0 = pos_v[2 * q, pl.ds(c * _LANES, _LANES)]
                    pv1 = pos_v[2 * q + 1, pl.ds(c * _LANES, _LANES)]
                    v0 = raw_v[2 * p, pl.ds(h0 + c * _LANES, _LANES)] + pv0
                    v1 = raw_v[2 * p + 1, pl.ds(h1 + c * _LANES, _LANES)] + pv1
                    raw_v[p, pl.ds(c * _LANES, _LANES)] = v0
                    raw_v[p, pl.ds(_NUM_HID + c * _LANES, _LANES)] = v1
                return carry
            lax.fori_loop(0, _MAXLEN // 2, srow, 0)

    gstart(0, idx0, hof0, raw0, g0)
    gstart(1, idx1, hof1, raw1, g1)

    def step(i, carry):
        for b, (idx_v, hof_v, raw_v, gsem, osem) in enumerate(bufs):
            g = 2 * i + b
            off2 = pl.multiple_of((base + g * _CHUNK) // 2, _PAIRS)
            gwait(idx_v, raw_v, gsem)
            compact(raw_v, hof_v)
            pltpu.async_copy(raw_v.at[pl.ds(0, _PAIRS)],
                             out_hbm.at[pl.ds(off2, _PAIRS)], osem)

            nxt = g + 2

            @pl.when(nxt < _NCHUNK)
            def _():
                pltpu.make_async_copy(raw_v.at[pl.ds(0, _PAIRS)],
                                      out_hbm.at[pl.ds(off2, _PAIRS)],
                                      osem).wait()
                gstart(nxt, idx_v, hof_v, raw_v, gsem)
        return carry

    lax.fori_loop(0, _NCHUNK // 2, step, 0)

    for idx_v, hof_v, raw_v, gsem, osem in bufs:
        pltpu.make_async_copy(raw_v.at[pl.ds(0, _PAIRS)],
                              out_hbm.at[pl.ds(0, _PAIRS)], osem).wait()


_mesh = plsc.VectorSubcoreMesh(core_axis_name="c", subcore_axis_name="s")

_tok_kernel = functools.partial(
    pl.kernel,
    mesh=_mesh,
    out_type=jax.ShapeDtypeStruct((_TOTAL // 2, 2 * _NUM_HID), jnp.float32),
    scratch_types=[
        pltpu.VMEM((_MAXLEN, _NUM_HID), jnp.float32),
        pltpu.VMEM((_CHUNK,), jnp.int32),
        pltpu.VMEM((_CHUNK,), jnp.int32),
        pltpu.VMEM((_CHUNK + _LANES,), jnp.int32),
        pltpu.VMEM((_CHUNK + _LANES,), jnp.int32),
        pltpu.VMEM((_CHUNK, 2 * _NUM_HID), jnp.float32),
        pltpu.VMEM((_CHUNK, 2 * _NUM_HID), jnp.float32),
        pltpu.SemaphoreType.DMA,
        pltpu.SemaphoreType.DMA,
        pltpu.SemaphoreType.DMA,
        pltpu.SemaphoreType.DMA,
    ],
)(_body)


@jax.jit
def kernel(x, emb_table, pos_table):
    x_flat = x.reshape(-1).astype(jnp.int32)
    idx2 = x_flat >> 1
    hoff = (x_flat & 1) * _NUM_HID
    emb2 = emb_table.reshape(_NUM_VOCAB // 2, 2 * _NUM_HID)
    out = _tok_kernel(idx2, hoff, emb2, pos_table)
    return out.reshape(_BATCH, _SEQ, _NUM_HID)


# speedup vs baseline: 1.3768x; 1.3768x over previous
import functools

import jax
import jax.numpy as jnp
from jax import lax
from jax.experimental import pallas as pl
from jax.experimental.pallas import tpu as pltpu
from jax.experimental.pallas import tpu_sc as plsc

_NUM_VOCAB = 1000000
_MAXLEN = 200
_NUM_HID = 64
_BATCH = 4096
_SEQ = 200

_NC = 2
_NS = 16
_NW = _NC * _NS
_TOTAL = _BATCH * _SEQ
_ROWS_PER_W = _TOTAL // _NW
_CHUNK = 800
_NCHUNK = _ROWS_PER_W // _CHUNK
_REPS = _CHUNK // _MAXLEN
_LANES = 16
_SLICES = _NUM_HID // _LANES


def _body(x_hbm, emb_hbm, pos_hbm, out_hbm,
          pos_v, idx0, idx1, tok0, tok1, g0, g1, o0, o1):
    cid = lax.axis_index("c")
    sid = lax.axis_index("s")
    wid = sid * _NC + cid
    base = pl.multiple_of(wid * _ROWS_PER_W, _CHUNK)

    pltpu.sync_copy(pos_hbm, pos_v)

    bufs = ((idx0, tok0, g0, o0), (idx1, tok1, g1, o1))

    def gstart(g, idx_v, tok_v, gsem):
        off = pl.multiple_of(base + g * _CHUNK, _CHUNK)
        pltpu.sync_copy(x_hbm.at[pl.ds(off, _CHUNK)], idx_v)
        pltpu.async_copy(emb_hbm.at[idx_v], tok_v, gsem)

    def gwait(idx_v, tok_v, gsem):
        pltpu.make_async_copy(emb_hbm.at[idx_v], tok_v, gsem).wait()

    def add_pos(tok_v):
        def srow(s, carry):
            for c in range(_SLICES):
                pv = pos_v[s, pl.ds(c * _LANES, _LANES)]
                for rep in range(_REPS):
                    plsc.addupdate(
                        tok_v.at[rep * _MAXLEN + s, pl.ds(c * _LANES, _LANES)],
                        pv)
            return carry
        lax.fori_loop(0, _MAXLEN, srow, 0)

    def ostart(g, tok_v, osem):
        row0 = (base + g * _CHUNK) // _MAXLEN
        for rep in range(_REPS):
            pltpu.async_copy(tok_v.at[pl.ds(rep * _MAXLEN, _MAXLEN)],
                             out_hbm.at[row0 + rep], osem)

    def owait(tok_v, osem):
        for rep in range(_REPS):
            pltpu.make_async_copy(tok_v.at[pl.ds(rep * _MAXLEN, _MAXLEN)],
                                  out_hbm.at[0], osem).wait()

    gstart(0, idx0, tok0, g0)
    gstart(1, idx1, tok1, g1)

    def step(i, carry):
        for b, (idx_v, tok_v, gsem, osem) in enumerate(bufs):
            g = 2 * i + b
            gwait(idx_v, tok_v, gsem)
            add_pos(tok_v)
            ostart(g, tok_v, osem)

            nxt = g + 2

            @pl.when(nxt < _NCHUNK)
            def _():
                owait(tok_v, osem)
                gstart(nxt, idx_v, tok_v, gsem)
        return carry

    lax.fori_loop(0, _NCHUNK // 2, step, 0)

    for idx_v, tok_v, gsem, osem in bufs:
        owait(tok_v, osem)


_mesh = plsc.VectorSubcoreMesh(core_axis_name="c", subcore_axis_name="s")

_tok_kernel = functools.partial(
    pl.kernel,
    mesh=_mesh,
    compiler_params=pltpu.CompilerParams(use_tc_tiling_on_sc=False),
    out_type=jax.ShapeDtypeStruct((_BATCH, _SEQ, _NUM_HID), jnp.float32),
    scratch_types=[
        pltpu.VMEM((_MAXLEN, _NUM_HID), jnp.float32),
        pltpu.VMEM((_CHUNK,), jnp.int32),
        pltpu.VMEM((_CHUNK,), jnp.int32),
        pltpu.VMEM((_CHUNK, _NUM_HID), jnp.float32),
        pltpu.VMEM((_CHUNK, _NUM_HID), jnp.float32),
        pltpu.SemaphoreType.DMA,
        pltpu.SemaphoreType.DMA,
        pltpu.SemaphoreType.DMA,
        pltpu.SemaphoreType.DMA,
    ],
)(_body)


@jax.jit
def kernel(x, emb_table, pos_table):
    return _tok_kernel(x.reshape(-1).astype(jnp.int32), emb_table, pos_table)
